# Initial kernel scaffold; baseline (speedup 1.0000x reference)
import jax
import jax.numpy as jnp
from jax.experimental import pallas as pl


def kernel(pred, gt):
    raise NotImplementedError("write your pallas kernel here")


# speedup vs baseline: 48.0487x; 48.0487x over previous
import functools

import jax
import jax.numpy as jnp
from jax import lax
from jax.experimental import pallas as pl
from jax.experimental.pallas import tpu as pltpu
from jax.experimental.pallas import tpu_sc as plsc

_B = 16384
_C = 1000
_K = _B + 1
_NBUCKET = 32768
_NC, _NS, _L = 2, 16, 16
_NW = _NC * _NS
_ROWS_W = _B // _NW
_CHUNK_ROWS = 32
_CHUNK = _CHUNK_ROWS * _C
_NCHUNK = _ROWS_W // _CHUNK_ROWS
_VPC = _CHUNK // _L


def _sc_body(pred_ref, gt_ref, cnt_out, sum_out, pl_out,
             chunk_v, gt_v, cnt_v, sum_v, pl_v):
    wid = lax.axis_index("s") * _NC + lax.axis_index("c")

    zi = jnp.zeros((_L,), jnp.int32)
    zf = jnp.zeros((_L,), jnp.float32)

    @pl.loop(0, _NBUCKET // _L, unroll=8)
    def _zero(i):
        cnt_v[pl.ds(i * _L, _L)] = zi
        sum_v[pl.ds(i * _L, _L)] = zf

    pltpu.sync_copy(gt_ref.at[pl.ds(wid * _ROWS_W, _ROWS_W)], gt_v)

    ones = jnp.ones((_L,), jnp.int32)
    pl_acc = zf
    base = wid * (_ROWS_W * _C)
    for c in range(_NCHUNK):
        pltpu.sync_copy(pred_ref.at[pl.ds(base + c * _CHUNK, _CHUNK)], chunk_v)

        @pl.loop(0, _VPC, unroll=8)
        def _hist(i):
            v = chunk_v[pl.ds(i * _L, _L)]
            bits = lax.bitcast_convert_type(v, jnp.int32) & jnp.int32(0x7FFFFFFF)
            bid = bits >> 16
            av = lax.bitcast_convert_type(bits, jnp.float32)
            plsc.addupdate_scatter(cnt_v, [bid], ones)
            plsc.addupdate_scatter(sum_v, [bid], av)

        for g in range(_CHUNK_ROWS // _L):
            rloc = lax.iota(jnp.int32, _L) + jnp.int32(g * _L)
            gts = gt_v[pl.ds(c * _CHUNK_ROWS + g * _L, _L)]
            fidx = rloc * jnp.int32(_C) + gts
            gv = plsc.load_gather(chunk_v, [fidx])
            gbits = lax.bitcast_convert_type(gv, jnp.int32) & jnp.int32(0x7FFFFFFF)
            gid = gbits >> 16
            ga = lax.bitcast_convert_type(gbits, jnp.float32)
            plsc.addupdate_scatter(cnt_v, [gid], -ones)
            plsc.addupdate_scatter(sum_v, [gid], -ga)
            pl_acc = pl_acc + jnp.abs(gv - 1.0)

    pl_v[...] = pl_acc
    pltpu.sync_copy(cnt_v, cnt_out.at[wid])
    pltpu.sync_copy(sum_v, sum_out.at[wid])
    pltpu.sync_copy(pl_v, pl_out.at[wid])


_sc_hist = functools.partial(
    pl.kernel,
    out_type=[
        jax.ShapeDtypeStruct((_NW, _NBUCKET), jnp.int32),
        jax.ShapeDtypeStruct((_NW, _NBUCKET), jnp.float32),
        jax.ShapeDtypeStruct((_NW, _L), jnp.float32),
    ],
    mesh=plsc.VectorSubcoreMesh(core_axis_name="c", subcore_axis_name="s"),
    compiler_params=pltpu.CompilerParams(needs_layout_passes=False),
    scratch_types=[
        pltpu.VMEM((_CHUNK,), jnp.float32),
        pltpu.VMEM((_ROWS_W,), jnp.int32),
        pltpu.VMEM((_NBUCKET,), jnp.int32),
        pltpu.VMEM((_NBUCKET,), jnp.float32),
        pltpu.VMEM((_L,), jnp.float32),
    ],
)(_sc_body)

_R = _NBUCKET // 128


def _finish_body(cnt_ref, sum_ref, plp_ref, out_ref):
    cnt = jnp.sum(cnt_ref[...], axis=0)
    sums = jnp.sum(sum_ref[...], axis=0)
    row = lax.broadcasted_iota(jnp.int32, (_R, 128), 0)
    col = lax.broadcasted_iota(jnp.int32, (_R, 128), 1)
    ids = row * 128 + col
    cnt = cnt + jnp.where(ids == 0, jnp.int32(_B), jnp.int32(0))

    def bs_body(_, lo_hi):
        lo, hi = lo_hi
        mid = (lo + hi + jnp.int32(1)) // 2
        c_ge = jnp.sum(jnp.where(ids >= mid, cnt, 0))
        ok = c_ge >= _K
        return (jnp.where(ok, mid, lo), jnp.where(ok, hi, mid - 1))

    lo, _hi = lax.fori_loop(
        0, 15, bs_body, (jnp.int32(0), jnp.int32(_NBUCKET - 1)))
    bstar = lo

    above = ids > bstar
    at_b = ids == bstar
    a_cnt = jnp.sum(jnp.where(above, cnt, 0))
    sum_above = jnp.sum(jnp.where(above, sums, 0.0))
    cnt_b = jnp.sum(jnp.where(at_b, cnt, 0))
    sum_b = jnp.sum(jnp.where(at_b, sums, 0.0))
    need = (jnp.int32(_K) - a_cnt).astype(jnp.float32)
    mean_b = sum_b / cnt_b.astype(jnp.float32)
    nl = (sum_above + need * mean_b) / jnp.float32(_K)
    pl_term = jnp.sum(plp_ref[...]) / jnp.float32(_B + 0.009)
    out_ref[...] = jnp.reshape((nl + pl_term) * 0.5, (1, 1))


_finish = pl.pallas_call(
    _finish_body,
    out_shape=jax.ShapeDtypeStruct((1, 1), jnp.float32),
)


def kernel(pred, gt):
    cnt, sums, plp = _sc_hist(pred.reshape(-1), gt)
    out = _finish(cnt.reshape(_NW, _R, 128), sums.reshape(_NW, _R, 128), plp)
    return out[0, 0]


# speedup vs baseline: 90.0613x; 1.8744x over previous
import functools

import jax
import jax.numpy as jnp
from jax import lax
from jax.experimental import pallas as pl
from jax.experimental.pallas import tpu as pltpu
from jax.experimental.pallas import tpu_sc as plsc

_B = 16384
_C = 1000
_K = _B + 1
_NBUCKET = 32768
_NC, _NS, _L = 2, 16, 16
_NW = _NC * _NS
_ROWS_W = _B // _NW
_CHUNK_ROWS = 16
_CHUNK = _CHUNK_ROWS * _C
_NCHUNK = _ROWS_W // _CHUNK_ROWS
_VPC = _CHUNK // _L
_BATCH = 8


def _sc_body(pred_ref, gt_ref, cnt_out, sum_out, pl_out,
             chunk0, chunk1, gt_v, cnt_v, sum_v, pl_v, sem0, sem1):
    wid = lax.axis_index("s") * _NC + lax.axis_index("c")

    zi = jnp.zeros((_L,), jnp.int32)
    zf = jnp.zeros((_L,), jnp.float32)

    @pl.loop(0, _NBUCKET // _L, unroll=8)
    def _zero(i):
        cnt_v[pl.ds(i * _L, _L)] = zi
        sum_v[pl.ds(i * _L, _L)] = zf

    pltpu.sync_copy(gt_ref.at[pl.ds(wid * _ROWS_W, _ROWS_W)], gt_v)

    ones = jnp.ones((_L,), jnp.int32)
    base = wid * (_ROWS_W * _C)
    bufs = (chunk0, chunk1)
    sems = (sem0, sem1)
    desc = [None, None]

    def issue(c):
        b = c % 2
        desc[b] = pltpu.async_copy(
            pred_ref.at[pl.ds(base + c * _CHUNK, _CHUNK)], bufs[b], sems[b])

    issue(0)
    pl_acc = zf
    for c in range(_NCHUNK):
        b = c % 2
        desc[b].wait()
        if c + 1 < _NCHUNK:
            issue(c + 1)
        buf = bufs[b]

        @plsc.parallel_loop(0, _VPC, step=_BATCH)
        def _hist(i):
            vs = [buf[pl.ds((i + j) * _L, _L)] for j in range(_BATCH)]
            bids, avs = [], []
            for v in vs:
                bits = (lax.bitcast_convert_type(v, jnp.int32)
                        & jnp.int32(0x7FFFFFFF))
                bids.append(bits >> 16)
                avs.append(lax.bitcast_convert_type(bits, jnp.float32))
            for bid, av in zip(bids, avs):
                plsc.addupdate_scatter(cnt_v, [bid], ones)
                plsc.addupdate_scatter(sum_v, [bid], av)

        rloc = lax.iota(jnp.int32, _L)
        gts = gt_v[pl.ds(c * _CHUNK_ROWS, _L)]
        fidx = rloc * jnp.int32(_C) + gts
        gv = plsc.load_gather(buf, [fidx])
        gbits = lax.bitcast_convert_type(gv, jnp.int32) & jnp.int32(0x7FFFFFFF)
        gid = gbits >> 16
        ga = lax.bitcast_convert_type(gbits, jnp.float32)
        plsc.addupdate_scatter(cnt_v, [gid], -ones)
        plsc.addupdate_scatter(sum_v, [gid], -ga)
        pl_acc = pl_acc + jnp.abs(gv - 1.0)

    pl_v[...] = pl_acc
    pltpu.sync_copy(cnt_v, cnt_out.at[wid])
    pltpu.sync_copy(sum_v, sum_out.at[wid])
    pltpu.sync_copy(pl_v, pl_out.at[wid])


_sc_hist = functools.partial(
    pl.kernel,
    out_type=[
        jax.ShapeDtypeStruct((_NW, _NBUCKET), jnp.int32),
        jax.ShapeDtypeStruct((_NW, _NBUCKET), jnp.float32),
        jax.ShapeDtypeStruct((_NW, _L), jnp.float32),
    ],
    mesh=plsc.VectorSubcoreMesh(core_axis_name="c", subcore_axis_name="s"),
    compiler_params=pltpu.CompilerParams(needs_layout_passes=False),
    scratch_types=[
        pltpu.VMEM((_CHUNK,), jnp.float32),
        pltpu.VMEM((_CHUNK,), jnp.float32),
        pltpu.VMEM((_ROWS_W,), jnp.int32),
        pltpu.VMEM((_NBUCKET,), jnp.int32),
        pltpu.VMEM((_NBUCKET,), jnp.float32),
        pltpu.VMEM((_L,), jnp.float32),
        pltpu.SemaphoreType.DMA,
        pltpu.SemaphoreType.DMA,
    ],
)(_sc_body)

_R = _NBUCKET // 128


def _finish_body(cnt_ref, sum_ref, plp_ref, out_ref):
    cnt = jnp.sum(cnt_ref[...], axis=0)
    sums = jnp.sum(sum_ref[...], axis=0)
    row = lax.broadcasted_iota(jnp.int32, (_R, 128), 0)
    col = lax.broadcasted_iota(jnp.int32, (_R, 128), 1)
    ids = row * 128 + col
    cnt = cnt + jnp.where(ids == 0, jnp.int32(_B), jnp.int32(0))

    def bs_body(_, lo_hi):
        lo, hi = lo_hi
        mid = (lo + hi + jnp.int32(1)) // 2
        c_ge = jnp.sum(jnp.where(ids >= mid, cnt, 0))
        ok = c_ge >= _K
        return (jnp.where(ok, mid, lo), jnp.where(ok, hi, mid - 1))

    lo, _hi = lax.fori_loop(
        0, 15, bs_body, (jnp.int32(0), jnp.int32(_NBUCKET - 1)))
    bstar = lo

    above = ids > bstar
    at_b = ids == bstar
    a_cnt = jnp.sum(jnp.where(above, cnt, 0))
    sum_above = jnp.sum(jnp.where(above, sums, 0.0))
    cnt_b = jnp.sum(jnp.where(at_b, cnt, 0))
    sum_b = jnp.sum(jnp.where(at_b, sums, 0.0))
    need = (jnp.int32(_K) - a_cnt).astype(jnp.float32)
    mean_b = sum_b / cnt_b.astype(jnp.float32)
    nl = (sum_above + need * mean_b) / jnp.float32(_K)
    pl_term = jnp.sum(plp_ref[...]) / jnp.float32(_B + 0.009)
    out_ref[...] = jnp.reshape((nl + pl_term) * 0.5, (1, 1))


_finish = pl.pallas_call(
    _finish_body,
    out_shape=jax.ShapeDtypeStruct((1, 1), jnp.float32),
)


def kernel(pred, gt):
    cnt, sums, plp = _sc_hist(pred.reshape(-1), gt)
    out = _finish(cnt.reshape(_NW, _R, 128), sums.reshape(_NW, _R, 128), plp)
    return out[0, 0]


# speedup vs baseline: 125.1780x; 1.3899x over previous
import functools

import jax
import jax.numpy as jnp
from jax import lax
from jax.experimental import pallas as pl
from jax.experimental.pallas import tpu as pltpu
from jax.experimental.pallas import tpu_sc as plsc

_B = 16384
_C = 1000
_K = _B + 1
_NBUCKET = 32768
_NC, _NS, _L = 2, 16, 16
_NW = _NC * _NS
_ROWS_W = _B // _NW
_CHUNK_ROWS = 16
_CHUNK = _CHUNK_ROWS * _C
_NCHUNK = _ROWS_W // _CHUNK_ROWS
_VPC = _CHUNK // _L
_BATCH = 8


def _sc_body(pred_ref, gt_ref, cnt_out, sum_out, pl_out,
             chunk0, chunk1, gt_v, cnt_v, sum_v, pl_v, sem0, sem1):
    wid = lax.axis_index("s") * _NC + lax.axis_index("c")

    zi = jnp.zeros((_L,), jnp.int32)
    zf = jnp.zeros((_L,), jnp.float32)

    @pl.loop(0, _NBUCKET // _L, unroll=8)
    def _zero(i):
        cnt_v[pl.ds(i * _L, _L)] = zi
        sum_v[pl.ds(i * _L, _L)] = zf

    pltpu.sync_copy(gt_ref.at[pl.ds(wid * _ROWS_W, _ROWS_W)], gt_v)

    ones = jnp.ones((_L,), jnp.int32)
    row0 = wid * _ROWS_W
    bufs = (chunk0, chunk1)
    sems = (sem0, sem1)

    def issue(cc, b):
        pltpu.async_copy(
            pred_ref.at[pl.ds(row0 + cc * _CHUNK_ROWS, _CHUNK_ROWS)],
            bufs[b], sems[b])

    issue(0, 0)
    issue(1, 1)
    pl_v[...] = zf
    tail_mask = lax.iota(jnp.int32, _L) >= jnp.int32(_L - _C % _L)
    rloc = lax.iota(jnp.int32, _L)

    @pl.loop(0, _NCHUNK, step=2)
    def _outer(c):
        for par in range(2):
            cc = c + par
            buf = bufs[par]
            pltpu.make_async_copy(
                pred_ref.at[pl.ds(row0 + cc * _CHUNK_ROWS, _CHUNK_ROWS)],
                buf, sems[par]).wait()

            @plsc.parallel_loop(0, _CHUNK_ROWS, step=1)
            def _hist(r):
                nfull = _C // _L
                vs = [buf[r, pl.ds(j * _L, _L)] for j in range(nfull)]
                vs.append(buf[r, pl.ds(_C - _L, _L)])
                for j, v in enumerate(vs):
                    bits = (lax.bitcast_convert_type(v, jnp.int32)
                            & jnp.int32(0x7FFFFFFF))
                    bid = bits >> 16
                    av = lax.bitcast_convert_type(bits, jnp.float32)
                    m = tail_mask if j == nfull else None
                    plsc.addupdate_scatter(cnt_v, [bid], ones, mask=m)
                    plsc.addupdate_scatter(sum_v, [bid], av, mask=m)

            gts = gt_v[pl.ds(cc * _CHUNK_ROWS, _L)]
            gv = plsc.load_gather(buf, [rloc, gts])
            gbits = (lax.bitcast_convert_type(gv, jnp.int32)
                     & jnp.int32(0x7FFFFFFF))
            gid = gbits >> 16
            ga = lax.bitcast_convert_type(gbits, jnp.float32)
            plsc.addupdate_scatter(cnt_v, [gid], -ones)
            plsc.addupdate_scatter(sum_v, [gid], -ga)
            pl_v[...] = pl_v[...] + jnp.abs(gv - 1.0)

            @pl.when(cc + 2 < _NCHUNK)
            def _prefetch():
                issue(cc + 2, par)
    pltpu.sync_copy(cnt_v, cnt_out.at[wid])
    pltpu.sync_copy(sum_v, sum_out.at[wid])
    pltpu.sync_copy(pl_v, pl_out.at[wid])


_sc_hist = functools.partial(
    pl.kernel,
    out_type=[
        jax.ShapeDtypeStruct((_NW, _NBUCKET), jnp.int32),
        jax.ShapeDtypeStruct((_NW, _NBUCKET), jnp.float32),
        jax.ShapeDtypeStruct((_NW, _L), jnp.float32),
    ],
    mesh=plsc.VectorSubcoreMesh(core_axis_name="c", subcore_axis_name="s"),
    compiler_params=pltpu.CompilerParams(needs_layout_passes=False),
    scratch_types=[
        pltpu.VMEM((_CHUNK_ROWS, _C), jnp.float32),
        pltpu.VMEM((_CHUNK_ROWS, _C), jnp.float32),
        pltpu.VMEM((_ROWS_W,), jnp.int32),
        pltpu.VMEM((_NBUCKET,), jnp.int32),
        pltpu.VMEM((_NBUCKET,), jnp.float32),
        pltpu.VMEM((_L,), jnp.float32),
        pltpu.SemaphoreType.DMA,
        pltpu.SemaphoreType.DMA,
    ],
)(_sc_body)

_R = _NBUCKET // 128


def _finish_body(cnt_ref, sum_ref, plp_ref, out_ref):
    cnt = jnp.sum(cnt_ref[...], axis=0)
    sums = jnp.sum(sum_ref[...], axis=0)
    row = lax.broadcasted_iota(jnp.int32, (_R, 128), 0)
    col = lax.broadcasted_iota(jnp.int32, (_R, 128), 1)
    ids = row * 128 + col
    cnt = cnt + jnp.where(ids == 0, jnp.int32(_B), jnp.int32(0))

    def bs_body(_, lo_hi):
        lo, hi = lo_hi
        mid = (lo + hi + jnp.int32(1)) // 2
        c_ge = jnp.sum(jnp.where(ids >= mid, cnt, 0))
        ok = c_ge >= _K
        return (jnp.where(ok, mid, lo), jnp.where(ok, hi, mid - 1))

    lo, _hi = lax.fori_loop(
        0, 15, bs_body, (jnp.int32(0), jnp.int32(_NBUCKET - 1)))
    bstar = lo

    above = ids > bstar
    at_b = ids == bstar
    a_cnt = jnp.sum(jnp.where(above, cnt, 0))
    sum_above = jnp.sum(jnp.where(above, sums, 0.0))
    cnt_b = jnp.sum(jnp.where(at_b, cnt, 0))
    sum_b = jnp.sum(jnp.where(at_b, sums, 0.0))
    need = (jnp.int32(_K) - a_cnt).astype(jnp.float32)
    mean_b = sum_b / cnt_b.astype(jnp.float32)
    nl = (sum_above + need * mean_b) / jnp.float32(_K)
    pl_term = jnp.sum(plp_ref[...]) / jnp.float32(_B + 0.009)
    out_ref[...] = jnp.reshape((nl + pl_term) * 0.5, (1, 1))


_finish = pl.pallas_call(
    _finish_body,
    out_shape=jax.ShapeDtypeStruct((1, 1), jnp.float32),
)


def kernel(pred, gt):
    cnt, sums, plp = _sc_hist(pred, gt)
    out = _finish(cnt.reshape(_NW, _R, 128), sums.reshape(_NW, _R, 128), plp)
    return out[0, 0]


# speedup vs baseline: 153.1303x; 1.2233x over previous
import functools

import jax
import jax.numpy as jnp
from jax import lax
from jax.experimental import pallas as pl
from jax.experimental.pallas import tpu as pltpu
from jax.experimental.pallas import tpu_sc as plsc

_B = 16384
_C = 1000
_K = _B + 1
_NBUCKET = 32768
_NC, _NS, _L = 2, 16, 16
_NW = _NC * _NS
_STRIP = _B // _NW
_GROUPS = _STRIP // _L
_CR = 40
_NCHUNK = _C // _CR
_BLK = 8


def _sc_body(ph_ref, gt_ref, cnt_out, sum_out, pl_out,
             chunk0, chunk1, gt_v, cnt_v, sum_v, pl_v, sem0, sem1):
    wid = lax.axis_index("s") * _NC + lax.axis_index("c")

    zi = jnp.zeros((_L,), jnp.int32)
    zf = jnp.zeros((_L,), jnp.float32)

    @pl.loop(0, _NBUCKET // _L, unroll=8)
    def _zero(i):
        cnt_v[pl.ds(i * _L, _L)] = zi
        sum_v[pl.ds(i * _L, _L)] = zf

    col0 = wid * _STRIP
    pltpu.sync_copy(gt_ref.at[pl.ds(col0, _STRIP)], gt_v)

    ones = jnp.ones((_L,), jnp.int32)
    onef = jnp.float32(1.0)
    lane = lax.iota(jnp.int32, _L)
    bufs = (chunk0, chunk1)
    sems = (sem0, sem1)

    def issue(cc, b):
        pltpu.async_copy(
            ph_ref.at[pl.ds(cc * _CR, _CR), pl.ds(col0, _STRIP)],
            bufs[b], sems[b])

    def process(cc, par):
        buf = bufs[par]
        pltpu.make_async_copy(
            ph_ref.at[pl.ds(cc * _CR, _CR), pl.ds(col0, _STRIP)],
            buf, sems[par]).wait()
        c0 = cc * _CR
        for blk in range(_GROUPS // _BLK):
            gts = [gt_v[pl.ds((blk * _BLK + j) * _L, _L)] for j in range(_BLK)]

            @plsc.parallel_loop(0, _CR, step=1)
            def _hist(r):
                for j in range(_BLK):
                    v = buf[r, pl.ds((blk * _BLK + j) * _L, _L)]
                    bits = (lax.bitcast_convert_type(v, jnp.int32)
                            & jnp.int32(0x7FFFFFFF))
                    bid = bits >> 16
                    av = lax.bitcast_convert_type(bits, jnp.float32)
                    plsc.addupdate_scatter(cnt_v, [bid], ones)
                    plsc.addupdate_scatter(sum_v, [bid], av)

            for j in range(_BLK):
                g = gts[j]
                inr = (g >= c0) & (g < c0 + _CR)
                gv = plsc.load_gather(
                    buf, [g - c0, lane + jnp.int32((blk * _BLK + j) * _L)],
                    mask=inr)
                gbits = (lax.bitcast_convert_type(gv, jnp.int32)
                         & jnp.int32(0x7FFFFFFF))
                gid = gbits >> 16
                ga = lax.bitcast_convert_type(gbits, jnp.float32)
                plsc.addupdate_scatter(cnt_v, [gid], -ones, mask=inr)
                plsc.addupdate_scatter(sum_v, [gid], -ga, mask=inr)
                pl_v[...] = pl_v[...] + jnp.where(
                    inr, jnp.abs(gv - onef), 0.0)

    issue(0, 0)
    issue(1, 1)
    pl_v[...] = zf

    @pl.loop(0, _NCHUNK - 1, step=2)
    def _outer(c):
        for par in range(2):
            cc = c + par
            process(cc, par)

            @pl.when(cc + 2 < _NCHUNK)
            def _prefetch():
                issue(cc + 2, par)

    process(_NCHUNK - 1, (_NCHUNK - 1) % 2)

    pltpu.sync_copy(cnt_v, cnt_out.at[wid])
    pltpu.sync_copy(sum_v, sum_out.at[wid])
    pltpu.sync_copy(pl_v, pl_out.at[wid])


_sc_hist = functools.partial(
    pl.kernel,
    out_type=[
        jax.ShapeDtypeStruct((_NW, _NBUCKET), jnp.int32),
        jax.ShapeDtypeStruct((_NW, _NBUCKET), jnp.float32),
        jax.ShapeDtypeStruct((_NW, _L), jnp.float32),
    ],
    mesh=plsc.VectorSubcoreMesh(core_axis_name="c", subcore_axis_name="s"),
    compiler_params=pltpu.CompilerParams(
        needs_layout_passes=False, use_tc_tiling_on_sc=True),
    scratch_types=[
        pltpu.VMEM((_CR, _STRIP), jnp.float32),
        pltpu.VMEM((_CR, _STRIP), jnp.float32),
        pltpu.VMEM((_STRIP,), jnp.int32),
        pltpu.VMEM((_NBUCKET,), jnp.int32),
        pltpu.VMEM((_NBUCKET,), jnp.float32),
        pltpu.VMEM((_L,), jnp.float32),
        pltpu.SemaphoreType.DMA,
        pltpu.SemaphoreType.DMA,
    ],
)(_sc_body)

_R = _NBUCKET // 128


def _finish_body(cnt_ref, sum_ref, plp_ref, out_ref):
    cnt = jnp.sum(cnt_ref[...], axis=0)
    sums = jnp.sum(sum_ref[...], axis=0)
    row = lax.broadcasted_iota(jnp.int32, (_R, 128), 0)
    col = lax.broadcasted_iota(jnp.int32, (_R, 128), 1)
    ids = row * 128 + col
    cnt = cnt + jnp.where(ids == 0, jnp.int32(_B), jnp.int32(0))

    def bs_body(_, lo_hi):
        lo, hi = lo_hi
        mid = (lo + hi + jnp.int32(1)) // 2
        c_ge = jnp.sum(jnp.where(ids >= mid, cnt, 0))
        ok = c_ge >= _K
        return (jnp.where(ok, mid, lo), jnp.where(ok, hi, mid - 1))

    lo, _hi = lax.fori_loop(
        0, 15, bs_body, (jnp.int32(0), jnp.int32(_NBUCKET - 1)))
    bstar = lo

    above = ids > bstar
    at_b = ids == bstar
    a_cnt = jnp.sum(jnp.where(above, cnt, 0))
    sum_above = jnp.sum(jnp.where(above, sums, 0.0))
    cnt_b = jnp.sum(jnp.where(at_b, cnt, 0))
    sum_b = jnp.sum(jnp.where(at_b, sums, 0.0))
    need = (jnp.int32(_K) - a_cnt).astype(jnp.float32)
    mean_b = sum_b / cnt_b.astype(jnp.float32)
    nl = (sum_above + need * mean_b) / jnp.float32(_K)
    pl_term = jnp.sum(plp_ref[...]) / jnp.float32(_B + 0.009)
    out_ref[...] = jnp.reshape((nl + pl_term) * 0.5, (1, 1))


_finish = pl.pallas_call(
    _finish_body,
    out_shape=jax.ShapeDtypeStruct((1, 1), jnp.float32),
)


def kernel(pred, gt):
    cnt, sums, plp = _sc_hist(pred.T, gt)
    out = _finish(cnt.reshape(_NW, _R, 128), sums.reshape(_NW, _R, 128), plp)
    return out[0, 0]
